# baseline (speedup 1.0000x reference)
import jax
import jax.numpy as jnp
from jax import lax
from jax.experimental import pallas as pl
from jax.experimental.pallas import tpu as pltpu

NUM_K = 1024
DIM = 256
TILE = 1024


def _vq_kernel(flat_ref, emb_ref, esq_ref, cs_ref,
               qst_ref, idx_ref, loss_ref, perp_ref, used_ref,
               counts_acc, loss_acc):
    i = pl.program_id(0)
    nsteps = pl.num_programs(0)

    f = flat_ref[...]
    emb = emb_ref[...]

    fsq = jnp.sum(f ** 2, axis=1, keepdims=True)
    fsq_row = fsq.T
    mm = lax.dot_general(emb, f, (((1,), (1,)), ((), ())),
                         preferred_element_type=jnp.float32)
    d2 = (fsq_row + esq_ref[...]) - 2.0 * mm

    mn_d2 = jnp.min(d2, axis=0, keepdims=True)
    p = jnp.maximum(mn_d2, 0.0)
    pb = lax.bitcast_convert_type(p, jnp.int32)
    sq_bits = []
    for j in range(9):
        xj = lax.bitcast_convert_type(pb + j, jnp.float32)
        sq_bits.append(lax.bitcast_convert_type(jnp.sqrt(xj), jnp.int32))
    base = sq_bits[0]
    for sb in sq_bits[1:]:
        base = jnp.minimum(base, sb)
    table = jnp.int32(7) << 27
    for j in range(9):
        oj = jnp.clip(sq_bits[j] - base, 0, 6)
        table = table | (oj << (3 * j))

    d2b = lax.bitcast_convert_type(d2, jnp.int32)
    r = jnp.clip(d2b - pb, 0, 9)
    rank = (jnp.right_shift(table, 3 * r)) & 7
    it = lax.broadcasted_iota(jnp.int32, (NUM_K, TILE), 0)
    key = (rank << 10) | it
    key_min = jnp.min(key, axis=0, keepdims=True)
    idx = key_min[0] & jnp.int32(1023)
    mn = lax.bitcast_convert_type(base + (key_min >> 10), jnp.float32)

    oh = (key == key_min).astype(jnp.float32)
    q = lax.dot_general(oh, emb, (((0,), (0,)), ((), ())),
                        preferred_element_type=jnp.float32)

    qst_ref[...] = q
    idx_ref[0, 0, :] = idx

    ones_row = jnp.ones((1, TILE), jnp.float32)
    tile_counts = lax.dot_general(ones_row, oh, (((1,), (1,)), ((), ())),
                                  preferred_element_type=jnp.float32)
    tile_loss = jnp.sum(mn * mn)

    @pl.when(i == 0)
    def _():
        counts_acc[...] = tile_counts
        loss_acc[0, 0] = tile_loss

    @pl.when(i > 0)
    def _():
        counts_acc[...] = counts_acc[...] + tile_counts
        loss_acc[0, 0] = loss_acc[0, 0] + tile_loss

    @pl.when(i == nsteps - 1)
    def _():
        n_tokens = jnp.float32(nsteps * TILE)
        avg = counts_acc[...] / n_tokens
        perp_ref[...] = jnp.exp(-jnp.sum(avg * jnp.log(avg + 1e-10))).reshape(1, 1)
        loss_ref[...] = (loss_acc[0, 0] / (n_tokens * jnp.float32(DIM))).reshape(1, 1)
        used_ref[...] = (jnp.sum((cs_ref[...] > 1e-05).astype(jnp.float32))
                         / jnp.float32(NUM_K)).reshape(1, 1)


def kernel(z, embedding, cluster_size):
    B, C, D, H, W = z.shape
    K, dim = embedding.shape
    n = B * D * H * W
    grid = n // TILE

    flat = jnp.transpose(z, (0, 2, 3, 4, 1)).reshape(-1, dim)
    esq = jnp.sum(embedding ** 2, axis=1)[:, None]

    qst_flat, idx3, loss, perp, used = pl.pallas_call(
        _vq_kernel,
        grid=(grid,),
        in_specs=[
            pl.BlockSpec((TILE, dim), lambda i: (i, 0)),
            pl.BlockSpec((K, dim), lambda i: (0, 0)),
            pl.BlockSpec((K, 1), lambda i: (0, 0)),
            pl.BlockSpec((1, K), lambda i: (0, 0)),
        ],
        out_specs=[
            pl.BlockSpec((TILE, dim), lambda i: (i, 0)),
            pl.BlockSpec((1, 1, TILE), lambda i: (i, 0, 0)),
            pl.BlockSpec((1, 1), lambda i: (0, 0)),
            pl.BlockSpec((1, 1), lambda i: (0, 0)),
            pl.BlockSpec((1, 1), lambda i: (0, 0)),
        ],
        out_shape=[
            jax.ShapeDtypeStruct((n, dim), jnp.float32),
            jax.ShapeDtypeStruct((grid, 1, TILE), jnp.int32),
            jax.ShapeDtypeStruct((1, 1), jnp.float32),
            jax.ShapeDtypeStruct((1, 1), jnp.float32),
            jax.ShapeDtypeStruct((1, 1), jnp.float32),
        ],
        scratch_shapes=[
            pltpu.VMEM((1, K), jnp.float32),
            pltpu.SMEM((1, 1), jnp.float32),
        ],
    )(flat, embedding, esq, cluster_size[None, :])

    quantized_st = jnp.transpose(qst_flat.reshape(B, D, H, W, C),
                                 (0, 4, 1, 2, 3))
    encoding_indices = idx3.reshape(B, D, H, W)
    return (quantized_st, loss.reshape(()), encoding_indices,
            perp.reshape(()), used.reshape(()))


# speedup vs baseline: 1.2461x; 1.2461x over previous
import jax
import jax.numpy as jnp
from jax import lax
from jax.experimental import pallas as pl
from jax.experimental.pallas import tpu as pltpu

NUM_K = 1024
DIM = 256
TILE = 1024


def _vq_kernel(flat_ref, emb_ref, esq_ref, cs_ref,
               qst_ref, idx_ref, loss_ref, perp_ref, used_ref,
               counts_acc, loss_acc):
    i = pl.program_id(0)
    nsteps = pl.num_programs(0)

    f = flat_ref[...]
    emb = emb_ref[...]

    fsq = jnp.sum(f ** 2, axis=1, keepdims=True)
    fsq_row = fsq.T
    mm = lax.dot_general(emb, f, (((1,), (1,)), ((), ())),
                         preferred_element_type=jnp.float32)
    d2 = (fsq_row + esq_ref[...]) - 2.0 * mm

    dist = jnp.sqrt(jnp.maximum(d2, 0.0))

    mn = jnp.min(dist, axis=0, keepdims=True)
    it = lax.broadcasted_iota(jnp.int32, (NUM_K, TILE), 0)
    idx = jnp.min(jnp.where(dist == mn, it, jnp.int32(1 << 30)), axis=0)

    oh = (it == idx[None, :]).astype(jnp.float32)
    q = lax.dot_general(oh, emb, (((0,), (0,)), ((), ())),
                        preferred_element_type=jnp.float32)

    qst_ref[...] = q
    idx_ref[0, 0, :] = idx

    ones_row = jnp.ones((1, TILE), jnp.float32)
    tile_counts = lax.dot_general(ones_row, oh, (((1,), (1,)), ((), ())),
                                  preferred_element_type=jnp.float32)
    tile_loss = jnp.sum(mn * mn)

    @pl.when(i == 0)
    def _():
        counts_acc[...] = tile_counts
        loss_acc[0, 0] = tile_loss

    @pl.when(i > 0)
    def _():
        counts_acc[...] = counts_acc[...] + tile_counts
        loss_acc[0, 0] = loss_acc[0, 0] + tile_loss

    @pl.when(i == nsteps - 1)
    def _():
        n_tokens = jnp.float32(nsteps * TILE)
        avg = counts_acc[...] / n_tokens
        perp_ref[...] = jnp.exp(-jnp.sum(avg * jnp.log(avg + 1e-10))).reshape(1, 1)
        loss_ref[...] = (loss_acc[0, 0] / (n_tokens * jnp.float32(DIM))).reshape(1, 1)
        used_ref[...] = (jnp.sum((cs_ref[...] > 1e-05).astype(jnp.float32))
                         / jnp.float32(NUM_K)).reshape(1, 1)


def kernel(z, embedding, cluster_size):
    B, C, D, H, W = z.shape
    K, dim = embedding.shape
    n = B * D * H * W
    grid = n // TILE

    flat = jnp.transpose(z, (0, 2, 3, 4, 1)).reshape(-1, dim)
    esq = jnp.sum(embedding ** 2, axis=1)[:, None]

    qst_flat, idx3, loss, perp, used = pl.pallas_call(
        _vq_kernel,
        grid=(grid,),
        in_specs=[
            pl.BlockSpec((TILE, dim), lambda i: (i, 0)),
            pl.BlockSpec((K, dim), lambda i: (0, 0)),
            pl.BlockSpec((K, 1), lambda i: (0, 0)),
            pl.BlockSpec((1, K), lambda i: (0, 0)),
        ],
        out_specs=[
            pl.BlockSpec((TILE, dim), lambda i: (i, 0)),
            pl.BlockSpec((1, 1, TILE), lambda i: (i, 0, 0)),
            pl.BlockSpec((1, 1), lambda i: (0, 0)),
            pl.BlockSpec((1, 1), lambda i: (0, 0)),
            pl.BlockSpec((1, 1), lambda i: (0, 0)),
        ],
        out_shape=[
            jax.ShapeDtypeStruct((n, dim), jnp.float32),
            jax.ShapeDtypeStruct((grid, 1, TILE), jnp.int32),
            jax.ShapeDtypeStruct((1, 1), jnp.float32),
            jax.ShapeDtypeStruct((1, 1), jnp.float32),
            jax.ShapeDtypeStruct((1, 1), jnp.float32),
        ],
        scratch_shapes=[
            pltpu.VMEM((1, K), jnp.float32),
            pltpu.SMEM((1, 1), jnp.float32),
        ],
    )(flat, embedding, esq, cluster_size[None, :])

    quantized_st = jnp.transpose(qst_flat.reshape(B, D, H, W, C),
                                 (0, 4, 1, 2, 3))
    encoding_indices = idx3.reshape(B, D, H, W)
    return (quantized_st, loss.reshape(()), encoding_indices,
            perp.reshape(()), used.reshape(()))


# speedup vs baseline: 1.2707x; 1.0197x over previous
import jax
import jax.numpy as jnp
from jax import lax
from jax.experimental import pallas as pl
from jax.experimental.pallas import tpu as pltpu

NUM_K = 1024
DIM = 256
TILE = 1024


def _vq_kernel(flat_ref, emb_ref, esq_ref, cs_ref,
               qst_ref, idx_ref, loss_ref, perp_ref, used_ref,
               counts_acc, loss_acc):
    i = pl.program_id(0)
    nsteps = pl.num_programs(0)

    f = flat_ref[...]
    emb = emb_ref[...]

    fsq = jnp.sum(f ** 2, axis=1, keepdims=True)
    mm = lax.dot_general(f, emb, (((1,), (1,)), ((), ())),
                         preferred_element_type=jnp.float32)
    d2 = (fsq + esq_ref[...]) - 2.0 * mm
    dist = jnp.sqrt(jnp.maximum(d2, 0.0))

    mn = jnp.min(dist, axis=1, keepdims=True)
    it = lax.broadcasted_iota(jnp.int32, (TILE, NUM_K), 1)
    idx = jnp.min(jnp.where(dist == mn, it, jnp.int32(1 << 30)), axis=1)

    oh = (it == idx[:, None]).astype(jnp.float32)
    q = lax.dot_general(oh, emb, (((1,), (0,)), ((), ())),
                        preferred_element_type=jnp.float32)

    qst_ref[...] = q
    idx_ref[0, 0, :] = idx

    ones_row = jnp.ones((1, TILE), jnp.float32)
    tile_counts = lax.dot_general(ones_row, oh, (((1,), (0,)), ((), ())),
                                  preferred_element_type=jnp.float32)
    tile_loss = jnp.sum(mn * mn)

    @pl.when(i == 0)
    def _():
        counts_acc[...] = tile_counts
        loss_acc[0, 0] = tile_loss

    @pl.when(i > 0)
    def _():
        counts_acc[...] = counts_acc[...] + tile_counts
        loss_acc[0, 0] = loss_acc[0, 0] + tile_loss

    @pl.when(i == nsteps - 1)
    def _():
        n_tokens = jnp.float32(nsteps * TILE)
        avg = counts_acc[...] / n_tokens
        perp_ref[...] = jnp.exp(-jnp.sum(avg * jnp.log(avg + 1e-10))).reshape(1, 1)
        loss_ref[...] = (loss_acc[0, 0] / (n_tokens * jnp.float32(DIM))).reshape(1, 1)
        used_ref[...] = (jnp.sum((cs_ref[...] > 1e-05).astype(jnp.float32))
                         / jnp.float32(NUM_K)).reshape(1, 1)


def kernel(z, embedding, cluster_size):
    B, C, D, H, W = z.shape
    K, dim = embedding.shape
    n = B * D * H * W
    grid = n // TILE

    flat = jnp.transpose(z, (0, 2, 3, 4, 1)).reshape(-1, dim)
    esq = jnp.sum(embedding ** 2, axis=1)[None, :]

    qst_flat, idx3, loss, perp, used = pl.pallas_call(
        _vq_kernel,
        grid=(grid,),
        in_specs=[
            pl.BlockSpec((TILE, dim), lambda i: (i, 0)),
            pl.BlockSpec((K, dim), lambda i: (0, 0)),
            pl.BlockSpec((1, K), lambda i: (0, 0)),
            pl.BlockSpec((1, K), lambda i: (0, 0)),
        ],
        out_specs=[
            pl.BlockSpec((TILE, dim), lambda i: (i, 0)),
            pl.BlockSpec((1, 1, TILE), lambda i: (i, 0, 0)),
            pl.BlockSpec((1, 1), lambda i: (0, 0)),
            pl.BlockSpec((1, 1), lambda i: (0, 0)),
            pl.BlockSpec((1, 1), lambda i: (0, 0)),
        ],
        out_shape=[
            jax.ShapeDtypeStruct((n, dim), jnp.float32),
            jax.ShapeDtypeStruct((grid, 1, TILE), jnp.int32),
            jax.ShapeDtypeStruct((1, 1), jnp.float32),
            jax.ShapeDtypeStruct((1, 1), jnp.float32),
            jax.ShapeDtypeStruct((1, 1), jnp.float32),
        ],
        scratch_shapes=[
            pltpu.VMEM((1, K), jnp.float32),
            pltpu.SMEM((1, 1), jnp.float32),
        ],
    )(flat, embedding, esq, cluster_size[None, :])

    quantized_st = jnp.transpose(qst_flat.reshape(B, D, H, W, C),
                                 (0, 4, 1, 2, 3))
    encoding_indices = idx3.reshape(B, D, H, W)
    return (quantized_st, loss.reshape(()), encoding_indices,
            perp.reshape(()), used.reshape(()))
